# baseline (speedup 1.0000x reference)
import jax
import jax.numpy as jnp
from jax.experimental import pallas as pl
from jax.experimental.pallas import tpu as pltpu

_TILE_M = 128
_TILE_N = 8192


def _cosine_logits_kernel(h_ref, w_ref, o_ref):
    h = h_ref[...]
    w = w_ref[...]
    hn = h * jax.lax.rsqrt(jnp.maximum(jnp.sum(h * h, axis=-1, keepdims=True), 1e-24))
    wn = w * jax.lax.rsqrt(jnp.maximum(jnp.sum(w * w, axis=-1, keepdims=True), 1e-24))
    o_ref[...] = jax.lax.dot_general(
        hn.astype(jnp.bfloat16), wn.astype(jnp.bfloat16),
        dimension_numbers=(((1,), (1,)), ((), ())),
        preferred_element_type=jnp.float32,
    )


def kernel(hidden, codebook):
    b, n, d = hidden.shape
    k, _ = codebook.shape
    m = b * n
    h2 = hidden.reshape(m, d)

    grid = (m // _TILE_M, k // _TILE_N)
    out = pl.pallas_call(
        _cosine_logits_kernel,
        grid=grid,
        in_specs=[
            pl.BlockSpec((_TILE_M, d), lambda i, j: (i, 0)),
            pl.BlockSpec((_TILE_N, d), lambda i, j: (j, 0)),
        ],
        out_specs=pl.BlockSpec((_TILE_M, _TILE_N), lambda i, j: (i, j)),
        out_shape=jax.ShapeDtypeStruct((m, k), jnp.float32),
        compiler_params=pltpu.CompilerParams(
            dimension_semantics=("parallel", "parallel"),
        ),
    )(h2, codebook)
    return out.reshape(b, n, k)


# speedup vs baseline: 1.2756x; 1.2756x over previous
import jax
import jax.numpy as jnp
from jax.experimental import pallas as pl
from jax.experimental.pallas import tpu as pltpu

_TILE_M = 256
_NBUF = 4


def _cosine_logits_kernel(h_ref, w_ref, o_ref, wn_ref, buf_ref, sems):
    w = w_ref[...]
    wn_ref[...] = (
        w * jax.lax.rsqrt(jnp.maximum(jnp.sum(w * w, axis=-1, keepdims=True), 1e-24))
    ).astype(jnp.bfloat16)

    m = h_ref.shape[0]
    n_tiles = m // _TILE_M

    def copy(i, slot):
        return pltpu.make_async_copy(
            buf_ref.at[slot],
            o_ref.at[pl.ds(i * _TILE_M, _TILE_M), :],
            sems.at[slot],
        )

    wn = wn_ref[...]
    for i in range(n_tiles):
        slot = i % _NBUF
        if i >= _NBUF:
            copy(i - _NBUF, slot).wait()
        h = h_ref[pl.ds(i * _TILE_M, _TILE_M), :]
        hn = h * jax.lax.rsqrt(
            jnp.maximum(jnp.sum(h * h, axis=-1, keepdims=True), 1e-24)
        )
        buf_ref[slot] = jax.lax.dot_general(
            hn.astype(jnp.bfloat16), wn,
            dimension_numbers=(((1,), (1,)), ((), ())),
            preferred_element_type=jnp.float32,
        )
        copy(i, slot).start()
    for i in range(max(n_tiles - _NBUF, 0), n_tiles):
        copy(i, i % _NBUF).wait()


def kernel(hidden, codebook):
    b, n, d = hidden.shape
    k, _ = codebook.shape
    m = b * n
    h2 = hidden.reshape(m, d)

    out = pl.pallas_call(
        _cosine_logits_kernel,
        in_specs=[
            pl.BlockSpec(memory_space=pltpu.VMEM),
            pl.BlockSpec(memory_space=pltpu.VMEM),
        ],
        out_specs=pl.BlockSpec(memory_space=pl.ANY),
        out_shape=jax.ShapeDtypeStruct((m, k), jnp.float32),
        scratch_shapes=[
            pltpu.VMEM((k, d), jnp.bfloat16),
            pltpu.VMEM((_NBUF, _TILE_M, k), jnp.float32),
            pltpu.SemaphoreType.DMA((_NBUF,)),
        ],
    )(h2, codebook)
    return out.reshape(b, n, k)


# speedup vs baseline: 1.3052x; 1.0232x over previous
import jax
import jax.numpy as jnp
from jax.experimental import pallas as pl
from jax.experimental.pallas import tpu as pltpu

_TILE_M = 288


def _cosine_logits_kernel(h_ref, w_ref, o_ref):
    h = h_ref[...]
    w = w_ref[...]
    hn = h * jax.lax.rsqrt(jnp.maximum(jnp.sum(h * h, axis=-1, keepdims=True), 1e-24))
    wn = w * jax.lax.rsqrt(jnp.maximum(jnp.sum(w * w, axis=-1, keepdims=True), 1e-24))
    o_ref[...] = jax.lax.dot_general(
        hn.astype(jnp.bfloat16), wn.astype(jnp.bfloat16),
        dimension_numbers=(((1,), (1,)), ((), ())),
        preferred_element_type=jnp.float32,
    )


def kernel(hidden, codebook):
    b, n, d = hidden.shape
    k, _ = codebook.shape
    m = b * n
    h2 = hidden.reshape(m, d)

    out = pl.pallas_call(
        _cosine_logits_kernel,
        grid=(m // _TILE_M,),
        in_specs=[
            pl.BlockSpec((_TILE_M, d), lambda i: (i, 0)),
            pl.BlockSpec((k, d), lambda i: (0, 0)),
        ],
        out_specs=pl.BlockSpec((_TILE_M, k), lambda i: (i, 0)),
        out_shape=jax.ShapeDtypeStruct((m, k), jnp.float32),
        compiler_params=pltpu.CompilerParams(
            dimension_semantics=("parallel",),
        ),
    )(h2, codebook)
    return out.reshape(b, n, k)
